# baseline (speedup 1.0000x reference)
import jax
import jax.numpy as jnp
from jax.experimental import pallas as pl

_K = 1024
_D = 256
_TILE = 256
_N = 12544
_NT = _N // _TILE


def _conv(x, w, b, stride, pad):
    out = jax.lax.conv_general_dilated(
        x, w, (stride, stride), ((pad, pad), (pad, pad)),
        dimension_numbers=('NCHW', 'OIHW', 'NCHW'))
    return out + b[None, :, None, None]


def _conv_t(x, w, b, stride, pad):
    k = w.shape[2]
    wt = jnp.transpose(jnp.flip(w, (2, 3)), (1, 0, 2, 3))
    p = k - 1 - pad
    out = jax.lax.conv_general_dilated(
        x, wt, (1, 1), ((p, p), (p, p)), lhs_dilation=(stride, stride),
        dimension_numbers=('NCHW', 'OIHW', 'NCHW'))
    return out + b[None, :, None, None]


def _bn(x, g, b, eps=1e-5):
    m = jnp.mean(x, axis=(0, 2, 3), keepdims=True)
    v = jnp.var(x, axis=(0, 2, 3), keepdims=True)
    return (x - m) / jnp.sqrt(v + eps) * g[None, :, None, None] + b[None, :, None, None]


def _vq_body(x_ref, emb_ref, a2_ref, b2_ref, closest_ref, q_ref):
    x = x_ref[...]
    emb = emb_ref[...]
    scores = jax.lax.dot_general(
        x, emb, (((1,), (1,)), ((), ())),
        preferred_element_type=jnp.float32)
    a2 = a2_ref[0, 0, :][:, None]
    b2 = b2_ref[...]
    d2 = a2 + b2 - 2.0 * scores
    dist = jnp.sqrt(jnp.maximum(d2, 0.0))
    mind = jnp.min(dist, axis=1, keepdims=True)
    kio = jax.lax.broadcasted_iota(jnp.int32, (_TILE, _K), 1)
    c = jnp.min(jnp.where(dist == mind, kio, _K), axis=1).astype(jnp.int32)
    closest_ref[0, 0, :] = c
    onehot = (kio == c[:, None]).astype(jnp.float32)
    q_ref[...] = jax.lax.dot_general(
        onehot, emb, (((1,), (0,)), ((), ())),
        preferred_element_type=jnp.float32,
        precision=jax.lax.Precision.HIGHEST)


def _pallas_vq(ops):
    xf, emb, a2f, b2r = ops
    return pl.pallas_call(
        _vq_body,
        grid=(_NT,),
        in_specs=[
            pl.BlockSpec((_TILE, _D), lambda i: (i, 0)),
            pl.BlockSpec((_K, _D), lambda i: (0, 0)),
            pl.BlockSpec((1, 1, _TILE), lambda i: (i, 0, 0)),
            pl.BlockSpec((1, _K), lambda i: (0, 0)),
        ],
        out_specs=[
            pl.BlockSpec((1, 1, _TILE), lambda i: (i, 0, 0)),
            pl.BlockSpec((_TILE, _D), lambda i: (i, 0)),
        ],
        out_shape=[
            jax.ShapeDtypeStruct((_NT, 1, _TILE), jnp.int32),
            jax.ShapeDtypeStruct((_N, _D), jnp.float32),
        ],
    )(xf, emb, a2f, b2r)


def kernel(x, params):
    p = params
    h = jax.nn.relu(_bn(_conv(x, p['e_conv1_w'], p['e_conv1_b'], 2, 1),
                        p['e_bn1_g'], p['e_bn1_b']))
    h = jax.nn.relu(_bn(_conv(h, p['e_conv2_w'], p['e_conv2_b'], 2, 1),
                        p['e_bn2_g'], p['e_bn2_b']))
    h = h + jax.nn.relu(_bn(_conv(h, p['e_res1_w'], p['e_res1_b'], 1, 1),
                            p['e_res1_bn_g'], p['e_res1_bn_b']))
    h = h + jax.nn.relu(_bn(_conv(h, p['e_res2_w'], p['e_res2_b'], 1, 0),
                            p['e_res2_bn_g'], p['e_res2_bn_b']))
    enc = _conv(h, p['e_proj_w'], p['e_proj_b'], 1, 0)
    B, C, H, W = enc.shape

    quant_input = enc.reshape(B, -1, C)
    emb = p['emb']
    a2 = jnp.sum(quant_input * quant_input, axis=-1, keepdims=True)
    b2 = jnp.sum(emb * emb, axis=-1)

    pred = x[0, 0, 0, 0] < 2.0

    def _xla_vq(ops):
        xf, embk, a2f, b2r = ops
        d2 = (a2f.reshape(-1, 1) + b2r
              - 2.0 * jnp.dot(xf, embk.T, preferred_element_type=jnp.float32))
        dists = jnp.sqrt(jnp.maximum(d2, 0.0))
        c = jnp.argmin(dists, axis=-1).astype(jnp.int32)
        return c.reshape(_NT, 1, _TILE), jnp.take(embk, c, axis=0)

    closest3, quantized = jax.lax.cond(
        pred,
        _pallas_vq,
        _xla_vq,
        (quant_input.reshape(-1, C), emb,
         a2.reshape(_NT, 1, _TILE), b2[None, :]))
    closest = closest3.reshape(B, -1)
    quantized3 = quantized.reshape(B, -1, _D)

    enc_flat = enc.reshape(B, -1, _D)
    commitment_loss = jnp.mean(
        (jax.lax.stop_gradient(quantized3) - enc_flat) ** 2)
    codebook_loss = jnp.mean(
        (quantized3 - jax.lax.stop_gradient(enc_flat)) ** 2)
    quantize_loss = codebook_loss + 0.255555 * commitment_loss
    quant_out = enc_flat + jax.lax.stop_gradient(quantized3 - enc_flat)
    quant_out = quant_out.reshape(B, C, H, W)

    h = _conv(quant_out, p['d_proj_w'], p['d_proj_b'], 1, 0)
    h = h + jax.nn.relu(_bn(_conv(h, p['d_res1_w'], p['d_res1_b'], 1, 1),
                            p['d_res1_bn_g'], p['d_res1_bn_b']))
    h = h + jax.nn.relu(_bn(_conv(h, p['d_res2_w'], p['d_res2_b'], 1, 1),
                            p['d_res2_bn_g'], p['d_res2_bn_b']))
    h = jax.nn.relu(_bn(_conv_t(h, p['d_ct1_w'], p['d_ct1_b'], 2, 1),
                        p['d_ct1_bn_g'], p['d_ct1_bn_b']))
    out = jax.nn.sigmoid(_conv_t(h, p['d_ct2_w'], p['d_ct2_b'], 2, 1))
    return out, closest, quantize_loss


# speedup vs baseline: 1.0172x; 1.0172x over previous
import jax
import jax.numpy as jnp
from jax.experimental import pallas as pl

_K = 1024
_D = 256
_TILE = 256
_N = 12544
_NT = _N // _TILE


def _conv(x, w, b, stride, pad):
    out = jax.lax.conv_general_dilated(
        x, w, (stride, stride), ((pad, pad), (pad, pad)),
        dimension_numbers=('NCHW', 'OIHW', 'NCHW'))
    return out + b[None, :, None, None]


def _conv_t(x, w, b, stride, pad):
    k = w.shape[2]
    wt = jnp.transpose(jnp.flip(w, (2, 3)), (1, 0, 2, 3))
    p = k - 1 - pad
    out = jax.lax.conv_general_dilated(
        x, wt, (1, 1), ((p, p), (p, p)), lhs_dilation=(stride, stride),
        dimension_numbers=('NCHW', 'OIHW', 'NCHW'))
    return out + b[None, :, None, None]


def _bn(x, g, b, eps=1e-5):
    m = jnp.mean(x, axis=(0, 2, 3), keepdims=True)
    v = jnp.var(x, axis=(0, 2, 3), keepdims=True)
    return (x - m) / jnp.sqrt(v + eps) * g[None, :, None, None] + b[None, :, None, None]


def _vq_body(x_ref, emb_ref, ehi_ref, emid_ref, elo_ref,
             a2_ref, b2_ref, closest_ref, q_ref):
    x = x_ref[...]
    emb = emb_ref[...]
    scores = jax.lax.dot_general(
        x, emb, (((1,), (1,)), ((), ())),
        preferred_element_type=jnp.float32)
    a2 = a2_ref[0, 0, :][:, None]
    b2 = b2_ref[...]
    d2 = a2 + b2 - 2.0 * scores
    dist = jnp.sqrt(jnp.maximum(d2, 0.0))
    mind = jnp.min(dist, axis=1, keepdims=True)
    kio = jax.lax.broadcasted_iota(jnp.int32, (_TILE, _K), 1)
    c = jnp.min(jnp.where(dist == mind, kio, _K), axis=1).astype(jnp.int32)
    closest_ref[0, 0, :] = c
    onehot = (kio == c[:, None]).astype(jnp.float32)
    def oh_dot(e):
        return jax.lax.dot_general(
            onehot, e, (((1,), (0,)), ((), ())),
            preferred_element_type=jnp.float32)
    q_ref[...] = (oh_dot(ehi_ref[...]) + oh_dot(emid_ref[...])) + oh_dot(elo_ref[...])


def _pallas_vq(ops):
    xf, emb, ehi, emid, elo, a2f, b2r = ops
    return pl.pallas_call(
        _vq_body,
        grid=(_NT,),
        in_specs=[
            pl.BlockSpec((_TILE, _D), lambda i: (i, 0)),
            pl.BlockSpec((_K, _D), lambda i: (0, 0)),
            pl.BlockSpec((_K, _D), lambda i: (0, 0)),
            pl.BlockSpec((_K, _D), lambda i: (0, 0)),
            pl.BlockSpec((_K, _D), lambda i: (0, 0)),
            pl.BlockSpec((1, 1, _TILE), lambda i: (i, 0, 0)),
            pl.BlockSpec((1, _K), lambda i: (0, 0)),
        ],
        out_specs=[
            pl.BlockSpec((1, 1, _TILE), lambda i: (i, 0, 0)),
            pl.BlockSpec((_TILE, _D), lambda i: (i, 0)),
        ],
        out_shape=[
            jax.ShapeDtypeStruct((_NT, 1, _TILE), jnp.int32),
            jax.ShapeDtypeStruct((_N, _D), jnp.float32),
        ],
    )(xf, emb, ehi, emid, elo, a2f, b2r)


def kernel(x, params):
    p = params
    h = jax.nn.relu(_bn(_conv(x, p['e_conv1_w'], p['e_conv1_b'], 2, 1),
                        p['e_bn1_g'], p['e_bn1_b']))
    h = jax.nn.relu(_bn(_conv(h, p['e_conv2_w'], p['e_conv2_b'], 2, 1),
                        p['e_bn2_g'], p['e_bn2_b']))
    h = h + jax.nn.relu(_bn(_conv(h, p['e_res1_w'], p['e_res1_b'], 1, 1),
                            p['e_res1_bn_g'], p['e_res1_bn_b']))
    h = h + jax.nn.relu(_bn(_conv(h, p['e_res2_w'], p['e_res2_b'], 1, 0),
                            p['e_res2_bn_g'], p['e_res2_bn_b']))
    enc = _conv(h, p['e_proj_w'], p['e_proj_b'], 1, 0)
    B, C, H, W = enc.shape

    quant_input = enc.reshape(B, -1, C)
    emb = p['emb']
    a2 = jnp.sum(quant_input * quant_input, axis=-1, keepdims=True)
    b2 = jnp.sum(emb * emb, axis=-1)

    pred = x[0, 0, 0, 0] < 2.0

    def _xla_vq(ops):
        xf, embk, _ehi, _emid, _elo, a2f, b2r = ops
        d2 = (a2f.reshape(-1, 1) + b2r
              - 2.0 * jnp.dot(xf, embk.T, preferred_element_type=jnp.float32))
        dists = jnp.sqrt(jnp.maximum(d2, 0.0))
        c = jnp.argmin(dists, axis=-1).astype(jnp.int32)
        return c.reshape(_NT, 1, _TILE), jnp.take(embk, c, axis=0)

    ehi = emb.astype(jnp.bfloat16).astype(jnp.float32)
    emid = (emb - ehi).astype(jnp.bfloat16).astype(jnp.float32)
    elo = (emb - ehi) - emid
    closest3, quantized = jax.lax.cond(
        pred,
        _pallas_vq,
        _xla_vq,
        (quant_input.reshape(-1, C), emb, ehi, emid, elo,
         a2.reshape(_NT, 1, _TILE), b2[None, :]))
    closest = closest3.reshape(B, -1)
    quantized3 = quantized.reshape(B, -1, _D)

    enc_flat = enc.reshape(B, -1, _D)
    commitment_loss = jnp.mean(
        (jax.lax.stop_gradient(quantized3) - enc_flat) ** 2)
    codebook_loss = jnp.mean(
        (quantized3 - jax.lax.stop_gradient(enc_flat)) ** 2)
    quantize_loss = codebook_loss + 0.255555 * commitment_loss
    quant_out = enc_flat + jax.lax.stop_gradient(quantized3 - enc_flat)
    quant_out = quant_out.reshape(B, C, H, W)

    h = _conv(quant_out, p['d_proj_w'], p['d_proj_b'], 1, 0)
    h = h + jax.nn.relu(_bn(_conv(h, p['d_res1_w'], p['d_res1_b'], 1, 1),
                            p['d_res1_bn_g'], p['d_res1_bn_b']))
    h = h + jax.nn.relu(_bn(_conv(h, p['d_res2_w'], p['d_res2_b'], 1, 1),
                            p['d_res2_bn_g'], p['d_res2_bn_b']))
    h = jax.nn.relu(_bn(_conv_t(h, p['d_ct1_w'], p['d_ct1_b'], 2, 1),
                        p['d_ct1_bn_g'], p['d_ct1_bn_b']))
    out = jax.nn.sigmoid(_conv_t(h, p['d_ct2_w'], p['d_ct2_b'], 2, 1))
    return out, closest, quantize_loss
